# baseline (speedup 1.0000x reference)
import functools

import jax
import jax.numpy as jnp
from jax import lax
from jax.experimental import pallas as pl
from jax.experimental.pallas import tpu as pltpu
from jax.experimental.pallas import tpu_sc as plsc

_NC = 2
_NS = 16
_NW = _NC * _NS
_CHUNK = 128
_LANES = 16


def _gather_body(n_blocks, seq, bsz, emb_dim, idx_hbm, table_hbm, out_hbm,
                 idx_v, rows_v, tr_v, isem, gsems, osems):
  wid = lax.axis_index("s") * _NC + lax.axis_index("c")
  nb_per_l = bsz // _CHUNK
  b0 = wid * n_blocks

  def blk(j):
    bid = b0 + j
    return bid // nb_per_l, bid % nb_per_l

  def stage(j, carry):
    l, c = blk(j)
    pltpu.make_async_copy(
        idx_hbm.at[l, pl.ds(c * _CHUNK, _CHUNK)], idx_v.at[j], isem).start()
    return carry

  lax.fori_loop(0, n_blocks, stage, 0, unroll=False)

  def drain_idx(j, carry):
    pltpu.make_async_copy(
        idx_hbm.at[0, pl.ds(0, _CHUNK)], idx_v.at[j], isem).wait()
    return carry

  lax.fori_loop(0, n_blocks, drain_idx, 0, unroll=False)

  def gather_copy(s, j):
    return pltpu.make_async_copy(table_hbm.at[idx_v.at[j]], rows_v.at[s],
                                 gsems[s])

  def transpose(s):
    rows_s = rows_v.at[s]
    tr_s = tr_v.at[s]
    row_idx = [lax.iota(jnp.int32, _LANES) + g * _LANES
               for g in range(_CHUNK // _LANES)]
    for d in range(emb_dim):
      col_d = jnp.full((_LANES,), d, jnp.int32)
      for g in range(_CHUNK // _LANES):
        vec = plsc.load_gather(rows_s, [row_idx[g], col_d])
        tr_s[d, pl.ds(g * _LANES, _LANES)] = vec

  def out_copy(s, j):
    l, c = blk(j)
    return pltpu.make_async_copy(
        tr_v.at[s], out_hbm.at[l, :, pl.ds(c * _CHUNK, _CHUNK)], osems[s])

  gather_copy(0, 0).start()
  gather_copy(1, 1).start()

  n_pairs = n_blocks // 2 - 1

  def body(p, carry):
    j = 2 * p
    gather_copy(0, j).wait()
    transpose(0)
    out_copy(0, j).start()
    gather_copy(1, j + 1).wait()
    transpose(1)
    out_copy(1, j + 1).start()
    out_copy(0, j).wait()
    gather_copy(0, j + 2).start()
    out_copy(1, j + 1).wait()
    gather_copy(1, j + 3).start()
    return carry

  lax.fori_loop(0, n_pairs, body, 0, unroll=False)

  j = 2 * n_pairs
  gather_copy(0, j).wait()
  transpose(0)
  out_copy(0, j).start()
  gather_copy(1, j + 1).wait()
  transpose(1)
  out_copy(1, j + 1).start()
  out_copy(0, j).wait()
  out_copy(1, j + 1).wait()


@functools.partial(jax.jit, static_argnames=("seq", "bsz", "emb_dim"))
def _sc_gather(idx, table, *, seq, bsz, emb_dim):
  mesh = plsc.VectorSubcoreMesh(
      core_axis_name="c", subcore_axis_name="s",
      num_cores=_NC, num_subcores=_NS)
  n_blocks = seq * bsz // (_NW * _CHUNK)
  run = pl.kernel(
      functools.partial(_gather_body, n_blocks, seq, bsz, emb_dim),
      out_type=jax.ShapeDtypeStruct((seq, emb_dim, bsz), jnp.float32),
      mesh=mesh,
      scratch_types=[
          pltpu.VMEM((n_blocks, _CHUNK), jnp.int32),
          pltpu.VMEM((2, _CHUNK, emb_dim), jnp.float32),
          pltpu.VMEM((2, emb_dim, _CHUNK), jnp.float32),
          pltpu.SemaphoreType.DMA,
          [pltpu.SemaphoreType.DMA] * 2,
          [pltpu.SemaphoreType.DMA] * 2,
      ],
      compiler_params=pltpu.CompilerParams(use_tc_tiling_on_sc=False,
                                           needs_layout_passes=False),
  )
  return run(idx, table)


def kernel(indices, table):
  bsz, seq = indices.shape
  vocab, emb_dim = table.shape
  idx_t = indices.T.astype(jnp.int32)
  out_t = _sc_gather(idx_t, table, seq=seq, bsz=bsz, emb_dim=emb_dim)
  return out_t.transpose(2, 0, 1)


# speedup vs baseline: 1.0802x; 1.0802x over previous
import functools

import jax
import jax.numpy as jnp
from jax import lax
from jax.experimental import pallas as pl
from jax.experimental.pallas import tpu as pltpu
from jax.experimental.pallas import tpu_sc as plsc

_NC = 2
_NS = 16
_NW = _NC * _NS
_CHUNK = 128
_LANES = 16


def _gather_body(n_blocks, seq, bsz, emb_dim, idx_hbm, table_hbm, out_hbm,
                 idx_v, rows_v, tr_v, isem, gsems, osems):
  wid = lax.axis_index("s") * _NC + lax.axis_index("c")
  nb_per_l = bsz // _CHUNK
  b0 = wid * n_blocks

  def blk(j):
    bid = b0 + j
    return bid // nb_per_l, bid % nb_per_l

  def stage(j, carry):
    l, c = blk(j)
    pltpu.make_async_copy(
        idx_hbm.at[l, pl.ds(c * _CHUNK, _CHUNK)], idx_v.at[j], isem).start()
    return carry

  lax.fori_loop(0, n_blocks, stage, 0, unroll=False)

  def drain_idx(j, carry):
    pltpu.make_async_copy(
        idx_hbm.at[0, pl.ds(0, _CHUNK)], idx_v.at[j], isem).wait()
    return carry

  lax.fori_loop(0, n_blocks, drain_idx, 0, unroll=False)

  def gather_copy(s, j):
    return pltpu.make_async_copy(table_hbm.at[idx_v.at[j]], rows_v.at[s],
                                 gsems[s])

  d_lo = lax.iota(jnp.int32, _LANES)
  d_hi = d_lo + _LANES

  def transpose(s):
    rows_s = rows_v.at[s]
    tr_s = tr_v.at[s]
    for t in range(_CHUNK):
      t_vec = jnp.full((_LANES,), t, jnp.int32)
      lo = rows_s[t, pl.ds(0, _LANES)]
      hi = rows_s[t, pl.ds(_LANES, _LANES)]
      plsc.store_scatter(tr_s, [d_lo, t_vec], lo)
      plsc.store_scatter(tr_s, [d_hi, t_vec], hi)

  def out_copy(s, j):
    l, c = blk(j)
    return pltpu.make_async_copy(
        tr_v.at[s], out_hbm.at[l, :, pl.ds(c * _CHUNK, _CHUNK)], osems[s])

  gather_copy(0, 0).start()
  gather_copy(1, 1).start()

  n_pairs = n_blocks // 2 - 1

  def body(p, carry):
    j = 2 * p
    gather_copy(0, j).wait()
    transpose(0)
    out_copy(0, j).start()
    gather_copy(1, j + 1).wait()
    transpose(1)
    out_copy(1, j + 1).start()
    out_copy(0, j).wait()
    gather_copy(0, j + 2).start()
    out_copy(1, j + 1).wait()
    gather_copy(1, j + 3).start()
    return carry

  lax.fori_loop(0, n_pairs, body, 0, unroll=False)

  j = 2 * n_pairs
  gather_copy(0, j).wait()
  transpose(0)
  out_copy(0, j).start()
  gather_copy(1, j + 1).wait()
  transpose(1)
  out_copy(1, j + 1).start()
  out_copy(0, j).wait()
  out_copy(1, j + 1).wait()


@functools.partial(jax.jit, static_argnames=("seq", "bsz", "emb_dim"))
def _sc_gather(idx, table, *, seq, bsz, emb_dim):
  mesh = plsc.VectorSubcoreMesh(
      core_axis_name="c", subcore_axis_name="s",
      num_cores=_NC, num_subcores=_NS)
  n_blocks = seq * bsz // (_NW * _CHUNK)
  run = pl.kernel(
      functools.partial(_gather_body, n_blocks, seq, bsz, emb_dim),
      out_type=jax.ShapeDtypeStruct((seq, emb_dim, bsz), jnp.float32),
      mesh=mesh,
      scratch_types=[
          pltpu.VMEM((n_blocks, _CHUNK), jnp.int32),
          pltpu.VMEM((2, _CHUNK, emb_dim), jnp.float32),
          pltpu.VMEM((2, emb_dim, _CHUNK), jnp.float32),
          pltpu.SemaphoreType.DMA,
          [pltpu.SemaphoreType.DMA] * 2,
          [pltpu.SemaphoreType.DMA] * 2,
      ],
      compiler_params=pltpu.CompilerParams(use_tc_tiling_on_sc=False,
                                           needs_layout_passes=False),
  )
  return run(idx, table)


def kernel(indices, table):
  bsz, seq = indices.shape
  vocab, emb_dim = table.shape
  idx_t = indices.T.astype(jnp.int32)
  out_t = _sc_gather(idx_t, table, seq=seq, bsz=bsz, emb_dim=emb_dim)
  return out_t.transpose(2, 0, 1)
